# baseline (speedup 1.0000x reference)
import functools

import jax
import jax.numpy as jnp
from jax import lax
from jax.experimental import pallas as pl
from jax.experimental.pallas import tpu as pltpu
from jax.experimental.pallas import tpu_sc as plsc

B = 4096
D = 1024
N = 8
C = 1000
CP = 1024
BN_EPS = 1e-3
LOG_EPS = 1e-30
BLK = 1024
NB = B // BLK

NW = 32
SPW = B // NW
VPW = SPW * N
ACC = N * CP


def _main_body(h_ref, m_ref, lab_ref, W_ref, b_ref, g_ref, be_ref,
               outh_ref, outr_ref, outwp_ref, outcnt_ref,
               s1, s2, cnt, hbuf):
    ph = pl.program_id(0)
    i = pl.program_id(1)

    @pl.when((ph == 0) & (i == 0))
    def _init():
        s1[...] = jnp.zeros_like(s1)
        s2[...] = jnp.zeros_like(s2)
        cnt[...] = jnp.zeros_like(cnt)

    @pl.when(ph == 0)
    def _stats():
        x = h_ref[...]
        hbuf[pl.ds(i * BLK, BLK), :] = x
        w = m_ref[pl.ds(i * BLK, BLK), :]
        wx = x * w
        s1[...] += jnp.sum(wx, axis=0, keepdims=True)
        s2[...] += jnp.sum(wx * x, axis=0, keepdims=True)
        cnt[...] += jnp.sum(w, axis=0, keepdims=True)

    @pl.when((ph == 1) & (i == 0))
    def _finalize_stats():
        denom = cnt[...] + 1e-8
        mean = s1[...] / denom
        var = s2[...] / denom - mean * mean
        scale = lax.rsqrt(var + BN_EPS) * g_ref[...]
        s1[...] = scale
        s2[...] = be_ref[...] - mean * scale
        outcnt_ref[...] = cnt[...]

    @pl.when(ph == 1)
    def _apply():
        x = hbuf[pl.ds(i * BLK, BLK), :]
        xn = x * s1[...] + s2[...]
        outh_ref[...] = xn
        act = jnp.dot(xn, W_ref[...], preferred_element_type=jnp.float32)
        act = act + b_ref[...]
        mx = jnp.max(act, axis=1, keepdims=True)
        e = jnp.exp(act - mx)
        p = e / jnp.sum(e, axis=1, keepdims=True)
        w = m_ref[pl.ds(i * BLK, BLK), :]
        outwp_ref[...] = p * w
        iota_n = lax.broadcasted_iota(jnp.int32, (BLK, N), 1)
        big = jnp.where(act == mx, iota_n, N)
        amin = jnp.min(big, axis=1, keepdims=True)
        outr_ref[...] = ((iota_n == amin) & (w > 0.5)).astype(jnp.int32)


_SC_MESH = plsc.VectorSubcoreMesh(core_axis_name="c", subcore_axis_name="s")


@functools.partial(
    pl.kernel,
    mesh=_SC_MESH,
    out_type=jax.ShapeDtypeStruct((NW, ACC), jnp.float32),
    compiler_params=pltpu.CompilerParams(needs_layout_passes=False),
    scratch_types=[
        pltpu.VMEM((VPW,), jnp.float32),
        pltpu.VMEM((SPW,), jnp.int32),
        pltpu.VMEM((ACC,), jnp.float32),
    ],
)
def _sc_pcn(wp_hbm, lab_hbm, out_hbm, wp_v, lab_v, acc_v):
    wid = lax.axis_index("s") * 2 + lax.axis_index("c")
    pltpu.sync_copy(wp_hbm.at[pl.ds(wid * VPW, VPW)], wp_v)
    pltpu.sync_copy(lab_hbm.at[pl.ds(wid * SPW, SPW)], lab_v)

    def _zero(k, carry):
        acc_v[pl.ds(pl.multiple_of(k * 16, 16), 16)] = jnp.zeros(
            (16,), jnp.float32)
        return carry
    lax.fori_loop(0, ACC // 16, _zero, 0)

    i16 = lax.broadcasted_iota(jnp.int32, (16,), 0)
    gate_off = (i16 & 7) * CP
    lo = i16 < 8
    hi = jnp.logical_not(lo)

    def _step(t, carry):
        vals = wp_v[pl.ds(pl.multiple_of(t * 16, 16), 16)]
        samp = t * 2 + (i16 >> 3)
        labs = plsc.load_gather(lab_v, [samp])
        tgt = gate_off + labs
        plsc.addupdate_scatter(acc_v, [tgt], vals, mask=lo)
        plsc.addupdate_scatter(acc_v, [tgt], vals, mask=hi)
        return carry
    lax.fori_loop(0, VPW // 16, _step, 0)

    pltpu.sync_copy(acc_v, out_hbm.at[wid])


def _entropy_body(p_ref, cnt_ref, outig_ref):
    tot = p_ref[pl.ds(0, N), :]
    for wloc in range(1, NW):
        tot = tot + p_ref[pl.ds(wloc * N, N), :]
    denom = cnt_ref[...] + 1e-8
    pcn = tot / denom
    pn = jnp.sum(pcn, axis=1, keepdims=True)
    pc = jnp.sum(pcn, axis=0, keepdims=True)
    ent_cn = -jnp.sum(pcn * jnp.log(pcn + LOG_EPS))
    ent_n = -jnp.sum(pn * jnp.log(pn + LOG_EPS))
    ent_c = -jnp.sum(pc * jnp.log(pc + LOG_EPS))
    outig_ref[...] = jnp.full((1, 1), -(ent_n + ent_c - ent_cn),
                              dtype=jnp.float32)


@jax.jit
def kernel(h_net, ig_mask, labels, W, b, gamma, beta):
    mask_f = ig_mask.astype(jnp.float32).reshape(B, 1)
    lab2d = labels.astype(jnp.int32).reshape(B, 1)
    h_normed, routing, wp, cntout = pl.pallas_call(
        _main_body,
        grid=(2, NB),
        in_specs=[
            pl.BlockSpec((BLK, D), lambda ph, i: (jnp.where(ph == 0, i, NB - 1), 0)),
            pl.BlockSpec((B, 1), lambda ph, i: (0, 0)),
            pl.BlockSpec((B, 1), lambda ph, i: (0, 0)),
            pl.BlockSpec((D, N), lambda ph, i: (0, 0)),
            pl.BlockSpec((1, N), lambda ph, i: (0, 0)),
            pl.BlockSpec((1, D), lambda ph, i: (0, 0)),
            pl.BlockSpec((1, D), lambda ph, i: (0, 0)),
        ],
        out_specs=[
            pl.BlockSpec((BLK, D), lambda ph, i: (jnp.where(ph == 0, 0, i), 0)),
            pl.BlockSpec((BLK, N), lambda ph, i: (jnp.where(ph == 0, 0, i), 0)),
            pl.BlockSpec((BLK, N), lambda ph, i: (jnp.where(ph == 0, 0, i), 0)),
            pl.BlockSpec((1, 1), lambda ph, i: (0, 0)),
        ],
        out_shape=[
            jax.ShapeDtypeStruct((B, D), jnp.float32),
            jax.ShapeDtypeStruct((B, N), jnp.int32),
            jax.ShapeDtypeStruct((B, N), jnp.float32),
            jax.ShapeDtypeStruct((1, 1), jnp.float32),
        ],
        scratch_shapes=[
            pltpu.VMEM((1, D), jnp.float32),
            pltpu.VMEM((1, D), jnp.float32),
            pltpu.VMEM((1, 1), jnp.float32),
            pltpu.VMEM((B, D), jnp.float32),
        ],
    )(h_net, mask_f, lab2d, W, b.reshape(1, N), gamma.reshape(1, D),
      beta.reshape(1, D))

    partials = _sc_pcn(wp.reshape(B * N), labels.astype(jnp.int32))

    ig = pl.pallas_call(
        _entropy_body,
        in_specs=[
            pl.BlockSpec((NW * N, CP), lambda: (0, 0)),
            pl.BlockSpec((1, 1), lambda: (0, 0)),
        ],
        out_specs=pl.BlockSpec((1, 1), lambda: (0, 0)),
        out_shape=jax.ShapeDtypeStruct((1, 1), jnp.float32),
    )(partials.reshape(NW * N, CP), cntout)

    return h_normed, ig[0, 0], routing


# speedup vs baseline: 1.0340x; 1.0340x over previous
import functools

import jax
import jax.numpy as jnp
from jax import lax
from jax.experimental import pallas as pl
from jax.experimental.pallas import tpu as pltpu
from jax.experimental.pallas import tpu_sc as plsc

B = 4096
D = 1024
N = 8
C = 1000
CP = 1024
BN_EPS = 1e-3
LOG_EPS = 1e-30
BLK = 1024
NB = B // BLK

NW = 32
SPW = B // NW
VPW = SPW * N
ACC = N * CP


def _main_body(h_ref, m_ref, lab_ref, W_ref, b_ref, g_ref, be_ref,
               outh_ref, outr_ref, outwp_ref, outcnt_ref,
               s1, s2, cnt, hbuf):
    ph = pl.program_id(0)
    i = pl.program_id(1)

    @pl.when((ph == 0) & (i == 0))
    def _init():
        s1[...] = jnp.zeros_like(s1)
        s2[...] = jnp.zeros_like(s2)
        cnt[...] = jnp.zeros_like(cnt)

    @pl.when(ph == 0)
    def _stats():
        x = h_ref[...]
        hbuf[pl.ds(i * BLK, BLK), :] = x
        w = m_ref[pl.ds(i * BLK, BLK), :]
        wx = x * w
        s1[...] += jnp.sum(wx, axis=0, keepdims=True)
        s2[...] += jnp.sum(wx * x, axis=0, keepdims=True)
        cnt[...] += jnp.sum(w, axis=0, keepdims=True)

    @pl.when((ph == 1) & (i == 0))
    def _finalize_stats():
        denom = cnt[...] + 1e-8
        mean = s1[...] / denom
        var = s2[...] / denom - mean * mean
        scale = lax.rsqrt(var + BN_EPS) * g_ref[...]
        s1[...] = scale
        s2[...] = be_ref[...] - mean * scale
        outcnt_ref[...] = cnt[...]

    @pl.when(ph == 1)
    def _apply():
        x = hbuf[pl.ds(i * BLK, BLK), :]
        xn = x * s1[...] + s2[...]
        outh_ref[...] = xn
        act = jnp.dot(xn, W_ref[...], preferred_element_type=jnp.float32)
        act = act + b_ref[...]
        mx = jnp.max(act, axis=1, keepdims=True)
        e = jnp.exp(act - mx)
        p = e / jnp.sum(e, axis=1, keepdims=True)
        w = m_ref[pl.ds(i * BLK, BLK), :]
        outwp_ref[...] = p * w
        iota_n = lax.broadcasted_iota(jnp.int32, (BLK, N), 1)
        big = jnp.where(act == mx, iota_n, N)
        amin = jnp.min(big, axis=1, keepdims=True)
        outr_ref[...] = ((iota_n == amin) & (w > 0.5)).astype(jnp.int32)


_SC_MESH = plsc.VectorSubcoreMesh(core_axis_name="c", subcore_axis_name="s")


@functools.partial(
    pl.kernel,
    mesh=_SC_MESH,
    out_type=jax.ShapeDtypeStruct((NW, ACC), jnp.float32),
    compiler_params=pltpu.CompilerParams(needs_layout_passes=False),
    scratch_types=[
        pltpu.VMEM((VPW,), jnp.float32),
        pltpu.VMEM((SPW,), jnp.int32),
        pltpu.VMEM((ACC,), jnp.float32),
    ],
)
def _sc_pcn(wp_hbm, lab_hbm, out_hbm, wp_v, lab_v, acc_v):
    wid = lax.axis_index("s") * 2 + lax.axis_index("c")
    pltpu.sync_copy(wp_hbm.at[pl.ds(wid * VPW, VPW)], wp_v)
    pltpu.sync_copy(lab_hbm.at[pl.ds(wid * SPW, SPW)], lab_v)

    ZU = 16
    zv = jnp.zeros((16,), jnp.float32)

    def _zero(k, carry):
        base = pl.multiple_of(k * (16 * ZU), 16)
        for u in range(ZU):
            acc_v[pl.ds(base + u * 16, 16)] = zv
        return carry
    lax.fori_loop(0, ACC // (16 * ZU), _zero, 0)

    i16 = lax.broadcasted_iota(jnp.int32, (16,), 0)
    gate_off = (i16 & 7) * CP
    lo = i16 < 8
    hi = jnp.logical_not(lo)

    MU = 8

    def _step(t, carry):
        for u in range(MU):
            tt = t * MU + u
            vals = wp_v[pl.ds(pl.multiple_of(tt * 16, 16), 16)]
            samp = tt * 2 + (i16 >> 3)
            labs = plsc.load_gather(lab_v, [samp])
            tgt = gate_off + labs
            plsc.addupdate_scatter(acc_v, [tgt], vals, mask=lo)
            plsc.addupdate_scatter(acc_v, [tgt], vals, mask=hi)
        return carry
    lax.fori_loop(0, VPW // 16 // MU, _step, 0)

    pltpu.sync_copy(acc_v, out_hbm.at[wid])


def _entropy_body(p_ref, cnt_ref, outig_ref):
    tot = p_ref[pl.ds(0, N), :]
    for wloc in range(1, NW):
        tot = tot + p_ref[pl.ds(wloc * N, N), :]
    denom = cnt_ref[...] + 1e-8
    pcn = tot / denom
    pn = jnp.sum(pcn, axis=1, keepdims=True)
    pc = jnp.sum(pcn, axis=0, keepdims=True)
    ent_cn = -jnp.sum(pcn * jnp.log(pcn + LOG_EPS))
    ent_n = -jnp.sum(pn * jnp.log(pn + LOG_EPS))
    ent_c = -jnp.sum(pc * jnp.log(pc + LOG_EPS))
    outig_ref[...] = jnp.full((1, 1), -(ent_n + ent_c - ent_cn),
                              dtype=jnp.float32)


@jax.jit
def kernel(h_net, ig_mask, labels, W, b, gamma, beta):
    mask_f = ig_mask.astype(jnp.float32).reshape(B, 1)
    lab2d = labels.astype(jnp.int32).reshape(B, 1)
    h_normed, routing, wp, cntout = pl.pallas_call(
        _main_body,
        grid=(2, NB),
        in_specs=[
            pl.BlockSpec((BLK, D), lambda ph, i: (jnp.where(ph == 0, i, NB - 1), 0)),
            pl.BlockSpec((B, 1), lambda ph, i: (0, 0)),
            pl.BlockSpec((B, 1), lambda ph, i: (0, 0)),
            pl.BlockSpec((D, N), lambda ph, i: (0, 0)),
            pl.BlockSpec((1, N), lambda ph, i: (0, 0)),
            pl.BlockSpec((1, D), lambda ph, i: (0, 0)),
            pl.BlockSpec((1, D), lambda ph, i: (0, 0)),
        ],
        out_specs=[
            pl.BlockSpec((BLK, D), lambda ph, i: (jnp.where(ph == 0, 0, i), 0)),
            pl.BlockSpec((BLK, N), lambda ph, i: (jnp.where(ph == 0, 0, i), 0)),
            pl.BlockSpec((BLK, N), lambda ph, i: (jnp.where(ph == 0, 0, i), 0)),
            pl.BlockSpec((1, 1), lambda ph, i: (0, 0)),
        ],
        out_shape=[
            jax.ShapeDtypeStruct((B, D), jnp.float32),
            jax.ShapeDtypeStruct((B, N), jnp.int32),
            jax.ShapeDtypeStruct((B, N), jnp.float32),
            jax.ShapeDtypeStruct((1, 1), jnp.float32),
        ],
        scratch_shapes=[
            pltpu.VMEM((1, D), jnp.float32),
            pltpu.VMEM((1, D), jnp.float32),
            pltpu.VMEM((1, 1), jnp.float32),
            pltpu.VMEM((B, D), jnp.float32),
        ],
    )(h_net, mask_f, lab2d, W, b.reshape(1, N), gamma.reshape(1, D),
      beta.reshape(1, D))

    partials = _sc_pcn(wp.reshape(B * N), labels.astype(jnp.int32))

    ig = pl.pallas_call(
        _entropy_body,
        in_specs=[
            pl.BlockSpec((NW * N, CP), lambda: (0, 0)),
            pl.BlockSpec((1, 1), lambda: (0, 0)),
        ],
        out_specs=pl.BlockSpec((1, 1), lambda: (0, 0)),
        out_shape=jax.ShapeDtypeStruct((1, 1), jnp.float32),
    )(partials.reshape(NW * N, CP), cntout)

    return h_normed, ig[0, 0], routing


# speedup vs baseline: 1.7843x; 1.7257x over previous
import jax
import jax.numpy as jnp
from jax import lax
from jax.experimental import pallas as pl
from jax.experimental.pallas import tpu as pltpu

B = 4096
D = 1024
N = 8
C = 1000
CP = 1024
BN_EPS = 1e-3
LOG_EPS = 1e-30
BLK = 2048
NB = B // BLK


def _body(h_ref, m_ref, lab_ref, W_ref, b_ref, g_ref, be_ref,
          outh_ref, outig_ref, outr_ref,
          s1, s2, cnt, pnc):
    ph = pl.program_id(0)
    i = pl.program_id(1)

    @pl.when((ph == 0) & (i == 0))
    def _init():
        s1[...] = jnp.zeros_like(s1)
        s2[...] = jnp.zeros_like(s2)
        cnt[...] = jnp.zeros_like(cnt)
        pnc[...] = jnp.zeros_like(pnc)

    @pl.when(ph == 0)
    def _stats():
        x = h_ref[...]
        w = m_ref[pl.ds(i * BLK, BLK), :]
        wx = x * w
        s1[...] += jnp.sum(wx, axis=0, keepdims=True)
        s2[...] += jnp.sum(wx * x, axis=0, keepdims=True)
        cnt[...] += jnp.sum(w, axis=0, keepdims=True)

    @pl.when((ph == 1) & (i == 0))
    def _finalize_stats():
        denom = cnt[...] + 1e-8
        mean = s1[...] / denom
        var = s2[...] / denom - mean * mean
        scale = lax.rsqrt(var + BN_EPS) * g_ref[...]
        s1[...] = scale
        s2[...] = be_ref[...] - mean * scale

    @pl.when(ph == 1)
    def _apply():
        x = h_ref[...]
        xn = x * s1[...] + s2[...]
        outh_ref[...] = xn
        act = jnp.dot(xn, W_ref[...], preferred_element_type=jnp.float32)
        act = act + b_ref[...]
        mx = jnp.max(act, axis=1, keepdims=True)
        e = jnp.exp(act - mx)
        p = e / jnp.sum(e, axis=1, keepdims=True)
        w = m_ref[pl.ds(i * BLK, BLK), :]
        wp = p * w
        lab = lab_ref[pl.ds(i * BLK, BLK), :]
        iota_c = lax.broadcasted_iota(jnp.int32, (BLK, CP), 1)
        onehot = (iota_c == lab).astype(jnp.bfloat16)
        pnc[...] += lax.dot_general(
            wp.astype(jnp.bfloat16), onehot, (((0,), (0,)), ((), ())),
            preferred_element_type=jnp.float32)
        iota_n = lax.broadcasted_iota(jnp.int32, (BLK, N), 1)
        big = jnp.where(act == mx, iota_n, N)
        amin = jnp.min(big, axis=1, keepdims=True)
        outr_ref[...] = ((iota_n == amin) & (w > 0.5)).astype(jnp.int32)

    @pl.when((ph == 1) & (i == NB - 1))
    def _entropy():
        denom = cnt[...] + 1e-8
        pcn = pnc[...] / denom
        pn = jnp.sum(pcn, axis=1, keepdims=True)
        pc = jnp.sum(pcn, axis=0, keepdims=True)
        ent_cn = -jnp.sum(pcn * jnp.log(pcn + LOG_EPS))
        ent_n = -jnp.sum(pn * jnp.log(pn + LOG_EPS))
        ent_c = -jnp.sum(pc * jnp.log(pc + LOG_EPS))
        outig_ref[...] = jnp.full((1, 1), -(ent_n + ent_c - ent_cn),
                                  dtype=jnp.float32)


@jax.jit
def kernel(h_net, ig_mask, labels, W, b, gamma, beta):
    mask_f = ig_mask.astype(jnp.float32).reshape(B, 1)
    lab = labels.astype(jnp.int32).reshape(B, 1)
    outs = pl.pallas_call(
        _body,
        grid=(2, NB),
        in_specs=[
            pl.BlockSpec((BLK, D), lambda ph, i: (i, 0)),
            pl.BlockSpec((B, 1), lambda ph, i: (0, 0)),
            pl.BlockSpec((B, 1), lambda ph, i: (0, 0)),
            pl.BlockSpec((D, N), lambda ph, i: (0, 0)),
            pl.BlockSpec((1, N), lambda ph, i: (0, 0)),
            pl.BlockSpec((1, D), lambda ph, i: (0, 0)),
            pl.BlockSpec((1, D), lambda ph, i: (0, 0)),
        ],
        out_specs=[
            pl.BlockSpec((BLK, D), lambda ph, i: (jnp.where(ph == 0, 0, i), 0)),
            pl.BlockSpec((1, 1), lambda ph, i: (0, 0)),
            pl.BlockSpec((BLK, N), lambda ph, i: (jnp.where(ph == 0, 0, i), 0)),
        ],
        out_shape=[
            jax.ShapeDtypeStruct((B, D), jnp.float32),
            jax.ShapeDtypeStruct((1, 1), jnp.float32),
            jax.ShapeDtypeStruct((B, N), jnp.int32),
        ],
        scratch_shapes=[
            pltpu.VMEM((1, D), jnp.float32),
            pltpu.VMEM((1, D), jnp.float32),
            pltpu.VMEM((1, 1), jnp.float32),
            pltpu.VMEM((N, CP), jnp.float32),
        ],
    )(h_net, mask_f, lab, W, b.reshape(1, N), gamma.reshape(1, D),
      beta.reshape(1, D))
    h_normed, ig, routing = outs
    return h_normed, ig[0, 0], routing
